# Initial kernel scaffold; baseline (speedup 1.0000x reference)
import functools

import jax
import jax.numpy as jnp
from jax import lax
from jax.experimental import pallas as pl
from jax.experimental.pallas import tpu as pltpu
from jax.experimental.pallas import tpu_sc as plsc

_N = 10000
_E = 320000
_F = 128
_HID = 128
_NC = 2
_NS = 16
_NW = _NC * _NS
_EW = _E // _NW
_CH = 80
_NQ = _EW // _CH
_ROWS_PER_TILE = 624

_sc_mesh = plsc.VectorSubcoreMesh(
    core_axis_name="c", subcore_axis_name="s", num_cores=_NC, num_subcores=_NS)

_f32 = jnp.float32
_i32 = jnp.int32


def _tc1_body(x_ref, wg_ref, att2_ref, we_ref, ae_ref, xw_ref, a2_ref, c_ref):
    xw = jnp.dot(x_ref[...], wg_ref[...], preferred_element_type=_f32)
    xw_ref[...] = xw
    a2_ref[...] = jnp.dot(xw, att2_ref[...], preferred_element_type=_f32)
    c_ref[...] = jnp.full((1, 128), jnp.sum(we_ref[...] * ae_ref[...]), _f32)


def _tc1(x, W_gat, att2, we, ae):
    blk = 2000
    return pl.pallas_call(
        _tc1_body,
        grid=(_N // blk,),
        in_specs=[
            pl.BlockSpec((blk, _F), lambda i: (i, 0)),
            pl.BlockSpec((_F, _HID), lambda i: (0, 0)),
            pl.BlockSpec((_HID, 2), lambda i: (0, 0)),
            pl.BlockSpec((1, _HID), lambda i: (0, 0)),
            pl.BlockSpec((1, _HID), lambda i: (0, 0)),
        ],
        out_specs=[
            pl.BlockSpec((blk, _HID), lambda i: (i, 0)),
            pl.BlockSpec((blk, 2), lambda i: (i, 0)),
            pl.BlockSpec((1, 128), lambda i: (0, 0)),
        ],
        out_shape=[
            jax.ShapeDtypeStruct((_N, _HID), _f32),
            jax.ShapeDtypeStruct((_N, 2), _f32),
            jax.ShapeDtypeStruct((1, 128), _f32),
        ],
    )(x, W_gat, att2, we, ae)


@functools.partial(
    pl.kernel,
    out_type=(
        jax.ShapeDtypeStruct((_E,), _f32),
        jax.ShapeDtypeStruct((_NC * _N,), _f32),
    ),
    mesh=_sc_mesh,
    scratch_types=[
        pltpu.VMEM((_N,), _f32),
        pltpu.VMEM((_N,), _f32),
        pltpu.VMEM((_NQ, _CH), _i32),
        pltpu.VMEM((_NQ, _CH), _i32),
        pltpu.VMEM((_EW,), _f32),
        pltpu.VMEM((_EW,), _f32),
        pltpu.VMEM((16,), _f32),
        pltpu.VMEM((640,), _f32),
        pltpu.VMEM_SHARED((_N,), _f32),
    ],
)
def _sc2(src_hbm, dst_hbm, ew_hbm, asrc_hbm, adst_hbm, c_hbm,
         ex_hbm, dpart_hbm,
         asrc_l, adst_l, src_l, dst_l, ew_l, ex_l, c_l, zbuf, denom_sp):
    cid = lax.axis_index("c")
    sid = lax.axis_index("s")
    wid = cid * _NS + sid
    base = wid * _EW

    pltpu.sync_copy(asrc_hbm, asrc_l)
    pltpu.sync_copy(adst_hbm, adst_l)
    pltpu.sync_copy(src_hbm.at[pl.ds(wid * _NQ, _NQ)], src_l)
    pltpu.sync_copy(dst_hbm.at[pl.ds(wid * _NQ, _NQ)], dst_l)
    pltpu.sync_copy(ew_hbm.at[pl.ds(base, _EW)], ew_l)
    pltpu.sync_copy(c_hbm, c_l)

    @pl.loop(0, 40)
    def _zero(i):
        zbuf[pl.ds(i * 16, 16)] = jnp.zeros((16,), _f32)

    pltpu.sync_copy(zbuf, denom_sp.at[pl.ds(sid * _ROWS_PER_TILE, 640)])
    plsc.subcore_barrier()

    cvec = c_l[...]

    @pl.loop(0, _NQ)
    def _chunk(q):
        @pl.loop(0, _CH // 16)
        def _grp(j):
            p = q * _CH + j * 16
            s16 = src_l[q, pl.ds(j * 16, 16)]
            d16 = dst_l[q, pl.ds(j * 16, 16)]
            al = (plsc.load_gather(asrc_l, [s16])
                  + plsc.load_gather(adst_l, [d16])
                  + ew_l[pl.ds(p, 16)] * cvec)
            al = jnp.maximum(al, 0.2 * al)
            ex_l[pl.ds(p, 16)] = jnp.exp(al)

        pltpu.sync_copy(ex_l.at[pl.ds(q * _CH, _CH)],
                        denom_sp.at[dst_l.at[q]], add=True)

    plsc.subcore_barrier()
    pltpu.sync_copy(ex_l, ex_hbm.at[pl.ds(base, _EW)])
    pltpu.sync_copy(denom_sp.at[pl.ds(sid * _ROWS_PER_TILE, 640)],
                    dpart_hbm.at[pl.ds(cid * _N + sid * _ROWS_PER_TILE, 640)])


def _tc3_body(dp_ref, rd_ref, dinv_ref):
    d = dp_ref[0:1, :] + dp_ref[1:2, :]
    rd = 1.0 / (d + 1e-16)
    rd_ref[...] = rd
    deg = d * rd
    dinv_ref[...] = jnp.where(deg > 0, lax.rsqrt(deg), 0.0)


def _tc3(dpart):
    return pl.pallas_call(
        _tc3_body,
        out_shape=[
            jax.ShapeDtypeStruct((1, _N), _f32),
            jax.ShapeDtypeStruct((1, _N), _f32),
        ],
    )(dpart)


@functools.partial(
    pl.kernel,
    out_type=(
        jax.ShapeDtypeStruct((_E,), _f32),
        jax.ShapeDtypeStruct((_E,), _f32),
        jax.ShapeDtypeStruct((_NC * _N, _HID), _f32),
    ),
    mesh=_sc_mesh,
    scratch_types=[
        pltpu.VMEM((_N,), _f32),
        pltpu.VMEM((_N,), _f32),
        pltpu.VMEM((_NQ, _CH), _i32),
        pltpu.VMEM((_NQ, _CH), _i32),
        pltpu.VMEM((_EW,), _f32),
        pltpu.VMEM((_EW,), _f32),
        pltpu.VMEM((_EW,), _f32),
        pltpu.VMEM((_CH, _HID), _f32),
        pltpu.VMEM((16, _HID), _f32),
        pltpu.VMEM_SHARED((_N, _HID), _f32),
    ],
)
def _sc4(src_hbm, dst_hbm, ex_hbm, rd_hbm, dinv_hbm, xw_hbm,
         attn_hbm, norm_hbm, hpart_hbm,
         rd_l, dinv_l, src_l, dst_l, ex_l, attn_l, norm_l, rows, zrows, h_sp):
    cid = lax.axis_index("c")
    sid = lax.axis_index("s")
    wid = cid * _NS + sid
    base = wid * _EW

    pltpu.sync_copy(rd_hbm, rd_l)
    pltpu.sync_copy(dinv_hbm, dinv_l)
    pltpu.sync_copy(src_hbm.at[pl.ds(wid * _NQ, _NQ)], src_l)
    pltpu.sync_copy(dst_hbm.at[pl.ds(wid * _NQ, _NQ)], dst_l)
    pltpu.sync_copy(ex_hbm.at[pl.ds(base, _EW)], ex_l)

    @pl.loop(0, 16)
    def _zr(i):
        for j in range(8):
            zrows[i, pl.ds(j * 16, 16)] = jnp.zeros((16,), _f32)

    @pl.loop(0, 40)
    def _zero(i):
        pltpu.sync_copy(zrows, h_sp.at[pl.ds(sid * _ROWS_PER_TILE + i * 16, 16)])

    plsc.subcore_barrier()

    @pl.loop(0, _NQ)
    def _chunk(q):
        pltpu.sync_copy(xw_hbm.at[src_l.at[q]], rows)

        @pl.loop(0, _CH // 16)
        def _grp(j):
            p = q * _CH + j * 16
            s16 = src_l[q, pl.ds(j * 16, 16)]
            d16 = dst_l[q, pl.ds(j * 16, 16)]
            at16 = ex_l[pl.ds(p, 16)] * plsc.load_gather(rd_l, [d16])
            attn_l[pl.ds(p, 16)] = at16
            norm_l[pl.ds(p, 16)] = (plsc.load_gather(dinv_l, [s16]) * at16
                                    * plsc.load_gather(dinv_l, [d16]))

        @pl.loop(0, _CH)
        def _scale(e):
            sp = plsc.load_gather(attn_l, [jnp.full((16,), q * _CH + e, _i32)])
            for j in range(8):
                rows[e, pl.ds(j * 16, 16)] = rows[e, pl.ds(j * 16, 16)] * sp

        pltpu.sync_copy(rows, h_sp.at[dst_l.at[q]], add=True)

    plsc.subcore_barrier()
    pltpu.sync_copy(attn_l, attn_hbm.at[pl.ds(base, _EW)])
    pltpu.sync_copy(norm_l, norm_hbm.at[pl.ds(base, _EW)])
    pltpu.sync_copy(h_sp.at[pl.ds(sid * _ROWS_PER_TILE, 640)],
                    hpart_hbm.at[pl.ds(cid * _N + sid * _ROWS_PER_TILE, 640)])


def _tc5_body(h0_ref, h1_ref, bg_ref, wg_ref, hw_ref):
    h = jnp.maximum(h0_ref[...] + h1_ref[...] + bg_ref[...], 0.0)
    hw_ref[...] = jnp.dot(h, wg_ref[...], preferred_element_type=_f32)


def _tc5(h0, h1, bg, W_gcn):
    blk = 2000
    return pl.pallas_call(
        _tc5_body,
        grid=(_N // blk,),
        in_specs=[
            pl.BlockSpec((blk, _HID), lambda i: (i, 0)),
            pl.BlockSpec((blk, _HID), lambda i: (i, 0)),
            pl.BlockSpec((1, _HID), lambda i: (0, 0)),
            pl.BlockSpec((_HID, _HID), lambda i: (0, 0)),
        ],
        out_specs=pl.BlockSpec((blk, _HID), lambda i: (i, 0)),
        out_shape=jax.ShapeDtypeStruct((_N, _HID), _f32),
    )(h0, h1, bg, W_gcn)


@functools.partial(
    pl.kernel,
    out_type=jax.ShapeDtypeStruct((_NC * _N, _HID), _f32),
    mesh=_sc_mesh,
    scratch_types=[
        pltpu.VMEM((_NQ, _CH), _i32),
        pltpu.VMEM((_NQ, _CH), _i32),
        pltpu.VMEM((_EW,), _f32),
        pltpu.VMEM((_CH, _HID), _f32),
        pltpu.VMEM((16, _HID), _f32),
        pltpu.VMEM_SHARED((_N, _HID), _f32),
    ],
)
def _sc6(src_hbm, dst_hbm, norm_hbm, hw_hbm, h2part_hbm,
         src_l, dst_l, norm_l, rows, zrows, h2_sp):
    cid = lax.axis_index("c")
    sid = lax.axis_index("s")
    wid = cid * _NS + sid
    base = wid * _EW

    pltpu.sync_copy(src_hbm.at[pl.ds(wid * _NQ, _NQ)], src_l)
    pltpu.sync_copy(dst_hbm.at[pl.ds(wid * _NQ, _NQ)], dst_l)
    pltpu.sync_copy(norm_hbm.at[pl.ds(base, _EW)], norm_l)

    @pl.loop(0, 16)
    def _zr(i):
        for j in range(8):
            zrows[i, pl.ds(j * 16, 16)] = jnp.zeros((16,), _f32)

    @pl.loop(0, 40)
    def _zero(i):
        pltpu.sync_copy(zrows, h2_sp.at[pl.ds(sid * _ROWS_PER_TILE + i * 16, 16)])

    plsc.subcore_barrier()

    @pl.loop(0, _NQ)
    def _chunk(q):
        pltpu.sync_copy(hw_hbm.at[src_l.at[q]], rows)

        @pl.loop(0, _CH)
        def _scale(e):
            sp = plsc.load_gather(norm_l, [jnp.full((16,), q * _CH + e, _i32)])
            for j in range(8):
                rows[e, pl.ds(j * 16, 16)] = rows[e, pl.ds(j * 16, 16)] * sp

        pltpu.sync_copy(rows, h2_sp.at[dst_l.at[q]], add=True)

    plsc.subcore_barrier()
    pltpu.sync_copy(h2_sp.at[pl.ds(sid * _ROWS_PER_TILE, 640)],
                    h2part_hbm.at[pl.ds(cid * _N + sid * _ROWS_PER_TILE, 640)])


def _tc7_body(h0_ref, h1_ref, bg_ref, wo_ref, bo_ref, out_ref):
    h2 = jnp.maximum(h0_ref[...] + h1_ref[...] + bg_ref[...], 0.0)
    out_ref[...] = jnp.dot(h2, wo_ref[...], preferred_element_type=_f32) + bo_ref[...]


def _tc7(h0, h1, bg, W_out, bo):
    blk = 2000
    nout = W_out.shape[1]
    return pl.pallas_call(
        _tc7_body,
        grid=(_N // blk,),
        in_specs=[
            pl.BlockSpec((blk, _HID), lambda i: (i, 0)),
            pl.BlockSpec((blk, _HID), lambda i: (i, 0)),
            pl.BlockSpec((1, _HID), lambda i: (0, 0)),
            pl.BlockSpec((_HID, nout), lambda i: (0, 0)),
            pl.BlockSpec((1, nout), lambda i: (0, 0)),
        ],
        out_specs=pl.BlockSpec((blk, nout), lambda i: (i, 0)),
        out_shape=jax.ShapeDtypeStruct((_N, nout), _f32),
    )(h0, h1, bg, W_out, bo)


def kernel(x, edge_index, edge_weight, W_gat, att_src, att_dst, W_edge,
           att_edge, b_gat, W_gcn, b_gcn, W_out, b_out):
    x = x.astype(_f32)
    edge_weight = edge_weight.astype(_f32)

    src2d = edge_index[0].reshape(_E // _CH, _CH)
    dst2d = edge_index[1].reshape(_E // _CH, _CH)
    ew = edge_weight.reshape(_E)
    att2 = jnp.concatenate([att_src, att_dst], axis=0).T

    xw, a2, cvec = _tc1(x, W_gat, att2, W_edge, att_edge)
    a_src = a2[:, 0]
    a_dst = a2[:, 1]
    c16 = cvec[0, :16]

    ex, dpart = _sc2(src2d, dst2d, ew, a_src, a_dst, c16)

    rd2, dinv2 = _tc3(dpart.reshape(_NC, _N))
    rdenom = rd2[0]
    dinv = dinv2[0]

    attn, norm, hpart = _sc4(src2d, dst2d, ex, rdenom, dinv, xw)

    hw = _tc5(hpart[:_N], hpart[_N:], b_gat.reshape(1, _HID), W_gcn)

    h2part = _sc6(src2d, dst2d, norm, hw)

    out = _tc7(h2part[:_N], h2part[_N:], b_gcn.reshape(1, _HID), W_out,
               b_out.reshape(1, -1))

    return (out, attn.reshape(_E, 1))


# speedup vs baseline: 5.5310x; 5.5310x over previous
import dataclasses
import functools

import jax
import jax.numpy as jnp
from jax import lax
from jax.experimental import pallas as pl
from jax.experimental.pallas import tpu as pltpu
from jax.experimental.pallas import tpu_sc as plsc

_N = 10000
_E = 320000
_F = 128
_HID = 128
_HH = _HID // 2
_NC = 2
_NS = 16
_NW = _NC * _NS
_EW = _E // _NW
_CH = 80
_NQ = _EW // _CH
_EP = _E // _NS
_NQP = _EP // _CH
_NP = 2
_NH = 2560
_NPAD = _NP * _NC * _NH
_TRASH = _NH
_AROWS = _NH + 8
_RPT = _NH // _NS
_RC = 32
_NRC = _RPT // _RC

_sc_mesh = plsc.VectorSubcoreMesh(
    core_axis_name="c", subcore_axis_name="s", num_cores=_NC, num_subcores=_NS)

_f32 = jnp.float32
_i32 = jnp.int32

_sc_params = pltpu.CompilerParams()
if "needs_layout_passes" in pltpu.CompilerParams.__dataclass_fields__:
    _sc_params = dataclasses.replace(_sc_params, needs_layout_passes=False)


def _tc1_body(x_ref, wg_ref, att2_ref, we_ref, ae_ref, xw_ref, a2_ref, c_ref):
    xw = jnp.dot(x_ref[...], wg_ref[...], preferred_element_type=_f32)
    xw_ref[...] = xw
    a2_ref[...] = jnp.dot(xw, att2_ref[...], preferred_element_type=_f32)
    c_ref[...] = jnp.full((1, 128), jnp.sum(we_ref[...] * ae_ref[...]), _f32)


def _tc1(x, W_gat, att2, we, ae):
    blk = 2000
    return pl.pallas_call(
        _tc1_body,
        grid=(_N // blk,),
        in_specs=[
            pl.BlockSpec((blk, _F), lambda i: (i, 0)),
            pl.BlockSpec((_F, _HID), lambda i: (0, 0)),
            pl.BlockSpec((_HID, 2), lambda i: (0, 0)),
            pl.BlockSpec((1, _HID), lambda i: (0, 0)),
            pl.BlockSpec((1, _HID), lambda i: (0, 0)),
        ],
        out_specs=[
            pl.BlockSpec((blk, _HID), lambda i: (i, 0)),
            pl.BlockSpec((blk, 2), lambda i: (i, 0)),
            pl.BlockSpec((1, 128), lambda i: (0, 0)),
        ],
        out_shape=[
            jax.ShapeDtypeStruct((_N, _HID), _f32),
            jax.ShapeDtypeStruct((_N, 2), _f32),
            jax.ShapeDtypeStruct((1, 128), _f32),
        ],
    )(x, W_gat, att2, we, ae)


@functools.partial(
    pl.kernel,
    out_type=(
        jax.ShapeDtypeStruct((_E,), _f32),
        jax.ShapeDtypeStruct((_NC * _N,), _f32),
    ),
    mesh=_sc_mesh,
    compiler_params=_sc_params,
    scratch_types=[
        pltpu.VMEM((_N,), _f32),
        pltpu.VMEM((_N,), _f32),
        pltpu.VMEM((_NQ, _CH), _i32),
        pltpu.VMEM((_NQ, _CH), _i32),
        pltpu.VMEM((_EW,), _f32),
        pltpu.VMEM((_EW,), _f32),
        pltpu.VMEM((16,), _f32),
        pltpu.VMEM((640,), _f32),
        pltpu.VMEM_SHARED((_N,), _f32),
    ],
)
def _sc2(src_hbm, dst_hbm, ew_hbm, asrc_hbm, adst_hbm, c_hbm,
         ex_hbm, dpart_hbm,
         asrc_l, adst_l, src_l, dst_l, ew_l, ex_l, c_l, zbuf, denom_sp):
    cid = lax.axis_index("c")
    sid = lax.axis_index("s")
    wid = cid * _NS + sid
    base = wid * _EW

    pltpu.sync_copy(asrc_hbm, asrc_l)
    pltpu.sync_copy(adst_hbm, adst_l)
    pltpu.sync_copy(src_hbm.at[wid], src_l)
    pltpu.sync_copy(dst_hbm.at[wid], dst_l)
    pltpu.sync_copy(ew_hbm.at[pl.ds(base, _EW)], ew_l)
    pltpu.sync_copy(c_hbm, c_l)

    @pl.loop(0, 40)
    def _zero(i):
        zbuf[pl.ds(i * 16, 16)] = jnp.zeros((16,), _f32)

    pltpu.sync_copy(zbuf, denom_sp.at[pl.ds(sid * 624, 640)])
    plsc.subcore_barrier()

    cvec = c_l[...]

    @pl.loop(0, _NQ)
    def _chunk(q):
        @pl.loop(0, _CH // 16)
        def _grp(j):
            p = q * _CH + j * 16
            s16 = src_l[q, pl.ds(j * 16, 16)]
            d16 = dst_l[q, pl.ds(j * 16, 16)]
            al = (plsc.load_gather(asrc_l, [s16])
                  + plsc.load_gather(adst_l, [d16])
                  + ew_l[pl.ds(p, 16)] * cvec)
            al = jnp.maximum(al, 0.2 * al)
            ex_l[pl.ds(p, 16)] = jnp.exp(al)

        pltpu.sync_copy(ex_l.at[pl.ds(q * _CH, _CH)],
                        denom_sp.at[dst_l.at[q]], add=True)

    plsc.subcore_barrier()
    pltpu.sync_copy(ex_l, ex_hbm.at[pl.ds(base, _EW)])
    pltpu.sync_copy(denom_sp.at[pl.ds(sid * 624, 640)], zbuf)
    pltpu.sync_copy(zbuf, dpart_hbm.at[pl.ds(cid * _N + sid * 624, 640)])


def _tc3_body(dp_ref, rd_ref, dinv_ref, sg_ref):
    d = dp_ref[0:1, :] + dp_ref[1:2, :]
    rd = 1.0 / (d + 1e-16)
    rd_ref[...] = rd
    deg = d * rd
    dinv = jnp.where(deg > 0, lax.rsqrt(deg), 0.0)
    dinv_ref[...] = dinv
    sg_ref[...] = dinv * rd


def _tc3(dpart):
    return pl.pallas_call(
        _tc3_body,
        out_shape=[
            jax.ShapeDtypeStruct((1, _N), _f32),
            jax.ShapeDtypeStruct((1, _N), _f32),
            jax.ShapeDtypeStruct((1, _N), _f32),
        ],
    )(dpart)


def _make_rowpass(with_scalars):
    out_type = [jax.ShapeDtypeStruct((_NPAD, _HID), _f32)]
    scratch = [
        pltpu.VMEM((_NQP, _CH), _i32),
        pltpu.VMEM((_NQP, _CH), _i32),
        pltpu.VMEM((1, _CH), _i32),
        pltpu.VMEM((_CH,), _f32),
        pltpu.VMEM((_NPAD,), _f32),
        pltpu.VMEM((_CH, _HID), _f32),
        pltpu.VMEM((_RC, _HID), _f32),
        pltpu.VMEM_SHARED((_AROWS, _HID), _f32),
    ]
    if with_scalars:
        out_type += [
            jax.ShapeDtypeStruct((_E,), _f32),
            jax.ShapeDtypeStruct((_E,), _f32),
        ]
        scratch += [
            pltpu.VMEM((_N,), _f32),
            pltpu.VMEM((_CH,), _f32),
            pltpu.VMEM((_CH,), _f32),
        ]

    def body(*refs):
        if with_scalars:
            (src_hbm, dst_hbm, es_hbm, mat_hbm, rscale_hbm, dinv_hbm,
             part_hbm, attn_hbm, w_hbm,
             src_l, dst_l, dstp_s, es_s, rs_l, rows, obuf, acc_sp,
             dinv_l, attn_s, w_s) = refs
        else:
            (src_hbm, dst_hbm, es_hbm, mat_hbm, rscale_hbm,
             part_hbm,
             src_l, dst_l, dstp_s, es_s, rs_l, rows, obuf, acc_sp) = refs
        cid = lax.axis_index("c")
        sid = lax.axis_index("s")
        ebase = sid * _EP

        pltpu.sync_copy(src_hbm.at[sid], src_l)
        pltpu.sync_copy(dst_hbm.at[sid], dst_l)
        pltpu.sync_copy(rscale_hbm, rs_l)
        if with_scalars:
            pltpu.sync_copy(dinv_hbm, dinv_l)

        if with_scalars:
            @pl.loop(0, _NQP)
            def _esc(q):
                pltpu.sync_copy(es_hbm.at[pl.ds(ebase + q * _CH, _CH)], es_s)

                @pl.loop(0, _CH // 16)
                def _g(j):
                    d16 = dst_l[q, pl.ds(j * 16, 16)]
                    s16 = src_l[q, pl.ds(j * 16, 16)]
                    ex16 = es_s[pl.ds(j * 16, 16)]
                    attn_s[pl.ds(j * 16, 16)] = (
                        ex16 * plsc.load_gather(rs_l, [d16]))
                    w_s[pl.ds(j * 16, 16)] = (
                        ex16 * plsc.load_gather(dinv_l, [s16]))

                @pl.when(cid == 0)
                def _wout():
                    pltpu.sync_copy(
                        attn_s, attn_hbm.at[pl.ds(ebase + q * _CH, _CH)])
                    pltpu.sync_copy(
                        w_s, w_hbm.at[pl.ds(ebase + q * _CH, _CH)])

        rbase0 = sid * _RPT

        for p in range(_NP):
            k = p * _NC + cid
            gbase = k * _NH

            @pl.loop(0, _RC)
            def _zr(i):
                @pl.loop(0, _HID // 16)
                def _zc(j):
                    obuf[i, pl.ds(j * 16, 16)] = jnp.zeros((16,), _f32)

            @pl.loop(0, _NRC)
            def _zero(i):
                pltpu.sync_copy(obuf, acc_sp.at[pl.ds(rbase0 + i * _RC, _RC)])

            @pl.when(sid == 0)
            def _zt():
                pltpu.sync_copy(obuf.at[pl.ds(0, 8)],
                                acc_sp.at[pl.ds(_TRASH, 8)])

            plsc.subcore_barrier()

            @pl.loop(0, _NQP)
            def _chunk(q):
                pltpu.sync_copy(mat_hbm.at[src_l.at[q]], rows)
                pltpu.sync_copy(es_hbm.at[pl.ds(ebase + q * _CH, _CH)], es_s)

                @pl.loop(0, _CH // 16)
                def _g(j):
                    d = dst_l[q, pl.ds(j * 16, 16)] - gbase
                    ok = (d >= 0) & (d < _NH)
                    dstp_s[0, pl.ds(j * 16, 16)] = jnp.where(ok, d, _TRASH)

                @pl.loop(0, _CH)
                def _scale(e):
                    sp = plsc.load_gather(es_s, [jnp.full((16,), e, _i32)])
                    for j in range(_HID // 16):
                        rows[e, pl.ds(j * 16, 16)] = (
                            rows[e, pl.ds(j * 16, 16)] * sp)

                pltpu.sync_copy(rows, acc_sp.at[dstp_s.at[0]], add=True)

            plsc.subcore_barrier()

            @pl.loop(0, _NRC)
            def _out(i):
                rbase = rbase0 + i * _RC
                pltpu.sync_copy(acc_sp.at[pl.ds(rbase, _RC)], obuf)

                @pl.loop(0, _RC)
                def _rs(r):
                    sp = plsc.load_gather(
                        rs_l, [jnp.full((16,), gbase + rbase + r, _i32)])
                    for j in range(_HID // 16):
                        obuf[r, pl.ds(j * 16, 16)] = (
                            obuf[r, pl.ds(j * 16, 16)] * sp)

                pltpu.sync_copy(obuf, part_hbm.at[pl.ds(gbase + rbase, _RC)])

            if p + 1 < _NP:
                plsc.subcore_barrier()

    return pl.kernel(
        body,
        out_type=tuple(out_type) if with_scalars else out_type[0],
        mesh=_sc_mesh,
        compiler_params=_sc_params,
        scratch_types=scratch,
    )


_rowpass_gat = _make_rowpass(True)
_rowpass_gcn = _make_rowpass(False)


def _tcmm_body(p_ref, b_ref, w_ref, bo_ref, o_ref):
    h = jnp.maximum(p_ref[...] + b_ref[...], 0.0)
    o_ref[...] = (jnp.dot(h, w_ref[...], preferred_element_type=_f32)
                  + bo_ref[...])


def _tcmm(p, b, w, bo):
    blk = 2000
    nout = w.shape[1]
    return pl.pallas_call(
        _tcmm_body,
        grid=(_N // blk,),
        in_specs=[
            pl.BlockSpec((blk, _HID), lambda i: (i, 0)),
            pl.BlockSpec((1, _HID), lambda i: (0, 0)),
            pl.BlockSpec((_HID, nout), lambda i: (0, 0)),
            pl.BlockSpec((1, nout), lambda i: (0, 0)),
        ],
        out_specs=pl.BlockSpec((blk, nout), lambda i: (i, 0)),
        out_shape=jax.ShapeDtypeStruct((_N, nout), _f32),
    )(p, b, w, bo)


def kernel(x, edge_index, edge_weight, W_gat, att_src, att_dst, W_edge,
           att_edge, b_gat, W_gcn, b_gcn, W_out, b_out):
    x = x.astype(_f32)
    edge_weight = edge_weight.astype(_f32)

    src32 = edge_index[0].reshape(_NW, _NQ, _CH)
    dst32 = edge_index[1].reshape(_NW, _NQ, _CH)
    src16 = edge_index[0].reshape(_NS, _NQP, _CH)
    dst16 = edge_index[1].reshape(_NS, _NQP, _CH)
    ew = edge_weight.reshape(_E)
    att2 = jnp.concatenate([att_src, att_dst], axis=0).T

    xw, a2, cvec = _tc1(x, W_gat, att2, W_edge, att_edge)
    a_src = a2[:, 0]
    a_dst = a2[:, 1]
    c16 = cvec[0, :16]

    ex, dpart = _sc2(src32, dst32, ew, a_src, a_dst, c16)

    rd2, dinv2, sg2 = _tc3(dpart.reshape(_NC, _N))
    rd = rd2[0]
    dinv = dinv2[0]
    sgcn = sg2[0]

    rd_pad = jnp.pad(rd, (0, _NPAD - _N))
    hpre, attn, w = _rowpass_gat(src16, dst16, ex, xw, rd_pad, dinv)

    hw = _tcmm(hpre[:_N], b_gat.reshape(1, _HID), W_gcn,
               jnp.zeros((1, _HID), _f32))

    sgcn_pad = jnp.pad(sgcn, (0, _NPAD - _N))
    h2pre = _rowpass_gcn(src16, dst16, w, hw, sgcn_pad)

    out = _tcmm(h2pre[:_N], b_gcn.reshape(1, _HID), W_out,
                b_out.reshape(1, -1))

    return (out, attn.reshape(_E, 1))


# speedup vs baseline: 9.7654x; 1.7656x over previous
import dataclasses
import functools

import jax
import jax.numpy as jnp
from jax import lax
from jax.experimental import pallas as pl
from jax.experimental.pallas import tpu as pltpu
from jax.experimental.pallas import tpu_sc as plsc

_N = 10000
_E = 320000
_F = 128
_HID = 128
_HH = _HID // 2
_NC = 2
_NS = 16
_NW = _NC * _NS
_EW = _E // _NW
_CH = 80
_NQ = _EW // _CH
_EP = _E // _NS
_NQP = _EP // _CH
_NP = 2
_NH = 2560
_NPAD = _NP * _NC * _NH
_TRASH = _NH
_AROWS = _NH + 8
_RPT = _NH // _NS
_RC = 16
_NRC = _RPT // _RC

_sc_mesh = plsc.VectorSubcoreMesh(
    core_axis_name="c", subcore_axis_name="s", num_cores=_NC, num_subcores=_NS)

_f32 = jnp.float32
_i32 = jnp.int32

_sc_params = pltpu.CompilerParams()
if "needs_layout_passes" in pltpu.CompilerParams.__dataclass_fields__:
    _sc_params = dataclasses.replace(_sc_params, needs_layout_passes=False)


def _tc1_body(x_ref, wg_ref, att2_ref, we_ref, ae_ref, xw_ref, a2_ref, c_ref):
    xw = jnp.dot(x_ref[...], wg_ref[...], preferred_element_type=_f32)
    xw_ref[...] = xw
    a2_ref[...] = jnp.dot(xw, att2_ref[...], preferred_element_type=_f32)
    c_ref[...] = jnp.full((1, 128), jnp.sum(we_ref[...] * ae_ref[...]), _f32)


def _tc1(x, W_gat, att2, we, ae):
    blk = 2000
    return pl.pallas_call(
        _tc1_body,
        grid=(_N // blk,),
        in_specs=[
            pl.BlockSpec((blk, _F), lambda i: (i, 0)),
            pl.BlockSpec((_F, _HID), lambda i: (0, 0)),
            pl.BlockSpec((_HID, 2), lambda i: (0, 0)),
            pl.BlockSpec((1, _HID), lambda i: (0, 0)),
            pl.BlockSpec((1, _HID), lambda i: (0, 0)),
        ],
        out_specs=[
            pl.BlockSpec((blk, _HID), lambda i: (i, 0)),
            pl.BlockSpec((blk, 2), lambda i: (i, 0)),
            pl.BlockSpec((1, 128), lambda i: (0, 0)),
        ],
        out_shape=[
            jax.ShapeDtypeStruct((_N, _HID), _f32),
            jax.ShapeDtypeStruct((_N, 2), _f32),
            jax.ShapeDtypeStruct((1, 128), _f32),
        ],
    )(x, W_gat, att2, we, ae)


@functools.partial(
    pl.kernel,
    out_type=(
        jax.ShapeDtypeStruct((_E,), _f32),
        jax.ShapeDtypeStruct((_NC * _N,), _f32),
    ),
    mesh=_sc_mesh,
    compiler_params=_sc_params,
    scratch_types=[
        pltpu.VMEM((_N,), _f32),
        pltpu.VMEM((_N,), _f32),
        pltpu.VMEM((_NQ, _CH), _i32),
        pltpu.VMEM((_NQ, _CH), _i32),
        pltpu.VMEM((_EW,), _f32),
        pltpu.VMEM((_EW,), _f32),
        pltpu.VMEM((16,), _f32),
        pltpu.VMEM((640,), _f32),
        pltpu.VMEM_SHARED((_N,), _f32),
    ],
)
def _sc2(src_hbm, dst_hbm, ew_hbm, asrc_hbm, adst_hbm, c_hbm,
         ex_hbm, dpart_hbm,
         asrc_l, adst_l, src_l, dst_l, ew_l, ex_l, c_l, zbuf, denom_sp):
    cid = lax.axis_index("c")
    sid = lax.axis_index("s")
    wid = cid * _NS + sid
    base = wid * _EW

    pltpu.sync_copy(asrc_hbm, asrc_l)
    pltpu.sync_copy(adst_hbm, adst_l)
    pltpu.sync_copy(src_hbm.at[wid], src_l)
    pltpu.sync_copy(dst_hbm.at[wid], dst_l)
    pltpu.sync_copy(ew_hbm.at[pl.ds(base, _EW)], ew_l)
    pltpu.sync_copy(c_hbm, c_l)

    @pl.loop(0, 40)
    def _zero(i):
        zbuf[pl.ds(i * 16, 16)] = jnp.zeros((16,), _f32)

    pltpu.sync_copy(zbuf, denom_sp.at[pl.ds(sid * 624, 640)])
    plsc.subcore_barrier()

    cvec = c_l[...]

    @pl.loop(0, _NQ)
    def _chunk(q):
        @pl.loop(0, _CH // 16)
        def _grp(j):
            p = q * _CH + j * 16
            s16 = src_l[q, pl.ds(j * 16, 16)]
            d16 = dst_l[q, pl.ds(j * 16, 16)]
            al = (plsc.load_gather(asrc_l, [s16])
                  + plsc.load_gather(adst_l, [d16])
                  + ew_l[pl.ds(p, 16)] * cvec)
            al = jnp.maximum(al, 0.2 * al)
            ex_l[pl.ds(p, 16)] = jnp.exp(al)

        pltpu.sync_copy(ex_l.at[pl.ds(q * _CH, _CH)],
                        denom_sp.at[dst_l.at[q]], add=True)

    plsc.subcore_barrier()
    pltpu.sync_copy(ex_l, ex_hbm.at[pl.ds(base, _EW)])
    pltpu.sync_copy(denom_sp.at[pl.ds(sid * 624, 640)], zbuf)
    pltpu.sync_copy(zbuf, dpart_hbm.at[pl.ds(cid * _N + sid * 624, 640)])


def _tc3_body(dp_ref, rd_ref, dinv_ref, sg_ref):
    d = dp_ref[0:1, :] + dp_ref[1:2, :]
    rd = 1.0 / (d + 1e-16)
    rd_ref[...] = rd
    deg = d * rd
    dinv = jnp.where(deg > 0, lax.rsqrt(deg), 0.0)
    dinv_ref[...] = dinv
    sg_ref[...] = dinv * rd


def _tc3(dpart):
    return pl.pallas_call(
        _tc3_body,
        out_shape=[
            jax.ShapeDtypeStruct((1, _N), _f32),
            jax.ShapeDtypeStruct((1, _N), _f32),
            jax.ShapeDtypeStruct((1, _N), _f32),
        ],
    )(dpart)


def _make_rowpass(with_scalars):
    out_type = [jax.ShapeDtypeStruct((_NPAD, _HID), _f32)]
    scratch = [
        pltpu.VMEM((_NQP, _CH), _i32),
        pltpu.VMEM((_NQP, _CH), _i32),
        pltpu.VMEM((1, _CH), _i32),
        pltpu.VMEM((_CH,), _f32),
        pltpu.VMEM((_CH,), _f32),
        pltpu.VMEM((_NPAD,), _f32),
        pltpu.VMEM((_CH, _HID), _f32),
        pltpu.VMEM((_CH, _HID), _f32),
        pltpu.VMEM((_RC, _HID), _f32),
        pltpu.VMEM_SHARED((_AROWS, _HID), _f32),
        pltpu.SemaphoreType.DMA,
        pltpu.SemaphoreType.DMA,
    ]
    if with_scalars:
        out_type += [
            jax.ShapeDtypeStruct((_E,), _f32),
            jax.ShapeDtypeStruct((_E,), _f32),
        ]
        scratch += [
            pltpu.VMEM((_N,), _f32),
            pltpu.VMEM((_CH,), _f32),
            pltpu.VMEM((_CH,), _f32),
        ]

    def body(*refs):
        if with_scalars:
            (src_hbm, dst_hbm, es_hbm, mat_hbm, rscale_hbm, dinv_hbm,
             part_hbm, attn_hbm, w_hbm,
             src_l, dst_l, dstp_s, es_a, es_b, rs_l, rows_a, rows_b,
             obuf, acc_sp, sem0, sem1, dinv_l, attn_s, w_s) = refs
        else:
            (src_hbm, dst_hbm, es_hbm, mat_hbm, rscale_hbm,
             part_hbm,
             src_l, dst_l, dstp_s, es_a, es_b, rs_l, rows_a, rows_b,
             obuf, acc_sp, sem0, sem1) = refs
        slots = ((rows_a, es_a, sem0), (rows_b, es_b, sem1))

        def _chunk_copies(q, b):
            rows, es, sem = slots[b]
            return (
                pltpu.make_async_copy(mat_hbm.at[src_l.at[q]], rows, sem),
                pltpu.make_async_copy(
                    es_hbm.at[pl.ds(ebase + q * _CH, _CH)], es, sem),
            )
        cid = lax.axis_index("c")
        sid = lax.axis_index("s")
        ebase = sid * _EP

        pltpu.sync_copy(src_hbm.at[sid], src_l)
        pltpu.sync_copy(dst_hbm.at[sid], dst_l)
        pltpu.sync_copy(rscale_hbm, rs_l)
        if with_scalars:
            pltpu.sync_copy(dinv_hbm, dinv_l)

        if with_scalars:
            @pl.loop(0, _NQP)
            def _esc(q):
                pltpu.sync_copy(es_hbm.at[pl.ds(ebase + q * _CH, _CH)], es_a)

                @pl.loop(0, _CH // 16)
                def _g(j):
                    d16 = dst_l[q, pl.ds(j * 16, 16)]
                    s16 = src_l[q, pl.ds(j * 16, 16)]
                    ex16 = es_a[pl.ds(j * 16, 16)]
                    attn_s[pl.ds(j * 16, 16)] = (
                        ex16 * plsc.load_gather(rs_l, [d16]))
                    w_s[pl.ds(j * 16, 16)] = (
                        ex16 * plsc.load_gather(dinv_l, [s16]))

                @pl.when(cid == 0)
                def _wout():
                    pltpu.sync_copy(
                        attn_s, attn_hbm.at[pl.ds(ebase + q * _CH, _CH)])
                    pltpu.sync_copy(
                        w_s, w_hbm.at[pl.ds(ebase + q * _CH, _CH)])

        rbase0 = sid * _RPT

        for p in range(_NP):
            k = p * _NC + cid
            gbase = k * _NH

            @pl.loop(0, _RC)
            def _zr(i):
                @pl.loop(0, _HID // 16)
                def _zc(j):
                    obuf[i, pl.ds(j * 16, 16)] = jnp.zeros((16,), _f32)

            @pl.loop(0, _NRC)
            def _zero(i):
                pltpu.sync_copy(obuf, acc_sp.at[pl.ds(rbase0 + i * _RC, _RC)])

            @pl.when(sid == 0)
            def _zt():
                pltpu.sync_copy(obuf.at[pl.ds(0, 8)],
                                acc_sp.at[pl.ds(_TRASH, 8)])

            for d in _chunk_copies(0, 0):
                d.start()

            plsc.subcore_barrier()

            @pl.loop(0, _NQP // 2)
            def _g2(g):
                for b in range(2):
                    q = g * 2 + b
                    rows, es, _ = slots[b]

                    @pl.when(q + 1 < _NQP)
                    def _nxt():
                        for d in _chunk_copies(q + 1, 1 - b):
                            d.start()

                    for d in _chunk_copies(q, b):
                        d.wait()

                    @pl.loop(0, _CH // 16)
                    def _g(j):
                        dd = dst_l[q, pl.ds(j * 16, 16)] - gbase
                        ok = (dd >= 0) & (dd < _NH)
                        dstp_s[0, pl.ds(j * 16, 16)] = jnp.where(ok, dd, _TRASH)

                    @pl.loop(0, _CH)
                    def _scale(e):
                        sp = plsc.load_gather(es, [jnp.full((16,), e, _i32)])
                        for j in range(_HID // 16):
                            rows[e, pl.ds(j * 16, 16)] = (
                                rows[e, pl.ds(j * 16, 16)] * sp)

                    pltpu.sync_copy(rows, acc_sp.at[dstp_s.at[0]], add=True)

            plsc.subcore_barrier()

            @pl.loop(0, _NRC)
            def _out(i):
                rbase = rbase0 + i * _RC
                pltpu.sync_copy(acc_sp.at[pl.ds(rbase, _RC)], obuf)

                @pl.loop(0, _RC)
                def _rs(r):
                    sp = plsc.load_gather(
                        rs_l, [jnp.full((16,), gbase + rbase + r, _i32)])
                    for j in range(_HID // 16):
                        obuf[r, pl.ds(j * 16, 16)] = (
                            obuf[r, pl.ds(j * 16, 16)] * sp)

                pltpu.sync_copy(obuf, part_hbm.at[pl.ds(gbase + rbase, _RC)])

            if p + 1 < _NP:
                plsc.subcore_barrier()

    return pl.kernel(
        body,
        out_type=tuple(out_type) if with_scalars else out_type[0],
        mesh=_sc_mesh,
        compiler_params=_sc_params,
        scratch_types=scratch,
    )


_rowpass_gat = _make_rowpass(True)
_rowpass_gcn = _make_rowpass(False)


def _tcmm_body(p_ref, b_ref, w_ref, bo_ref, o_ref):
    h = jnp.maximum(p_ref[...] + b_ref[...], 0.0)
    o_ref[...] = (jnp.dot(h, w_ref[...], preferred_element_type=_f32)
                  + bo_ref[...])


def _tcmm(p, b, w, bo):
    blk = 2000
    nout = w.shape[1]
    return pl.pallas_call(
        _tcmm_body,
        grid=(_N // blk,),
        in_specs=[
            pl.BlockSpec((blk, _HID), lambda i: (i, 0)),
            pl.BlockSpec((1, _HID), lambda i: (0, 0)),
            pl.BlockSpec((_HID, nout), lambda i: (0, 0)),
            pl.BlockSpec((1, nout), lambda i: (0, 0)),
        ],
        out_specs=pl.BlockSpec((blk, nout), lambda i: (i, 0)),
        out_shape=jax.ShapeDtypeStruct((_N, nout), _f32),
    )(p, b, w, bo)


def kernel(x, edge_index, edge_weight, W_gat, att_src, att_dst, W_edge,
           att_edge, b_gat, W_gcn, b_gcn, W_out, b_out):
    x = x.astype(_f32)
    edge_weight = edge_weight.astype(_f32)

    src32 = edge_index[0].reshape(_NW, _NQ, _CH)
    dst32 = edge_index[1].reshape(_NW, _NQ, _CH)
    src16 = edge_index[0].reshape(_NS, _NQP, _CH)
    dst16 = edge_index[1].reshape(_NS, _NQP, _CH)
    ew = edge_weight.reshape(_E)
    att2 = jnp.concatenate([att_src, att_dst], axis=0).T

    xw, a2, cvec = _tc1(x, W_gat, att2, W_edge, att_edge)
    a_src = a2[:, 0]
    a_dst = a2[:, 1]
    c16 = cvec[0, :16]

    ex, dpart = _sc2(src32, dst32, ew, a_src, a_dst, c16)

    rd2, dinv2, sg2 = _tc3(dpart.reshape(_NC, _N))
    rd = rd2[0]
    dinv = dinv2[0]
    sgcn = sg2[0]

    rd_pad = jnp.pad(rd, (0, _NPAD - _N))
    hpre, attn, w = _rowpass_gat(src16, dst16, ex, xw, rd_pad, dinv)

    hw = _tcmm(hpre[:_N], b_gat.reshape(1, _HID), W_gcn,
               jnp.zeros((1, _HID), _f32))

    sgcn_pad = jnp.pad(sgcn, (0, _NPAD - _N))
    h2pre = _rowpass_gcn(src16, dst16, w, hw, sgcn_pad)

    out = _tcmm(h2pre[:_N], b_gcn.reshape(1, _HID), W_out,
                b_out.reshape(1, -1))

    return (out, attn.reshape(_E, 1))


# speedup vs baseline: 9.7680x; 1.0003x over previous
import dataclasses
import functools

import jax
import jax.numpy as jnp
from jax import lax
from jax.experimental import pallas as pl
from jax.experimental.pallas import tpu as pltpu
from jax.experimental.pallas import tpu_sc as plsc

_N = 10000
_E = 320000
_F = 128
_HID = 128
_HH = _HID // 2
_NC = 2
_NS = 16
_NW = _NC * _NS
_EW = _E // _NW
_CH = 80
_NQ = _EW // _CH
_EP = _E // _NS
_NQP = _EP // _CH
_NP = 2
_NH = 2560
_NPAD = _NP * _NC * _NH
_TRASH = _NH
_AROWS = _NH + 8
_RPT = _NH // _NS
_RC = 16
_NRC = _RPT // _RC

_sc_mesh = plsc.VectorSubcoreMesh(
    core_axis_name="c", subcore_axis_name="s", num_cores=_NC, num_subcores=_NS)

_f32 = jnp.float32
_i32 = jnp.int32

_sc_params = pltpu.CompilerParams()
if "needs_layout_passes" in pltpu.CompilerParams.__dataclass_fields__:
    _sc_params = dataclasses.replace(_sc_params, needs_layout_passes=False)


def _tc1_body(x_ref, wg_ref, att2_ref, we_ref, ae_ref, xw_ref, a2_ref, c_ref):
    xw = jnp.dot(x_ref[...], wg_ref[...], preferred_element_type=_f32)
    xw_ref[...] = xw
    a2_ref[...] = jnp.dot(xw, att2_ref[...], preferred_element_type=_f32)
    c_ref[...] = jnp.full((1, 128), jnp.sum(we_ref[...] * ae_ref[...]), _f32)


def _tc1(x, W_gat, att2, we, ae):
    blk = 2000
    return pl.pallas_call(
        _tc1_body,
        grid=(_N // blk,),
        in_specs=[
            pl.BlockSpec((blk, _F), lambda i: (i, 0)),
            pl.BlockSpec((_F, _HID), lambda i: (0, 0)),
            pl.BlockSpec((_HID, 2), lambda i: (0, 0)),
            pl.BlockSpec((1, _HID), lambda i: (0, 0)),
            pl.BlockSpec((1, _HID), lambda i: (0, 0)),
        ],
        out_specs=[
            pl.BlockSpec((blk, _HID), lambda i: (i, 0)),
            pl.BlockSpec((blk, 2), lambda i: (i, 0)),
            pl.BlockSpec((1, 128), lambda i: (0, 0)),
        ],
        out_shape=[
            jax.ShapeDtypeStruct((_N, _HID), _f32),
            jax.ShapeDtypeStruct((_N, 2), _f32),
            jax.ShapeDtypeStruct((1, 128), _f32),
        ],
    )(x, W_gat, att2, we, ae)


@functools.partial(
    pl.kernel,
    out_type=(
        jax.ShapeDtypeStruct((_E,), _f32),
        jax.ShapeDtypeStruct((_NC * _N,), _f32),
    ),
    mesh=_sc_mesh,
    compiler_params=_sc_params,
    scratch_types=[
        pltpu.VMEM((_N,), _f32),
        pltpu.VMEM((_N,), _f32),
        pltpu.VMEM((_NQ, _CH), _i32),
        pltpu.VMEM((_NQ, _CH), _i32),
        pltpu.VMEM((_EW,), _f32),
        pltpu.VMEM((_EW,), _f32),
        pltpu.VMEM((16,), _f32),
        pltpu.VMEM((640,), _f32),
        pltpu.VMEM_SHARED((_N,), _f32),
    ],
)
def _sc2(src_hbm, dst_hbm, ew_hbm, asrc_hbm, adst_hbm, c_hbm,
         ex_hbm, dpart_hbm,
         asrc_l, adst_l, src_l, dst_l, ew_l, ex_l, c_l, zbuf, denom_sp):
    cid = lax.axis_index("c")
    sid = lax.axis_index("s")
    wid = cid * _NS + sid
    base = wid * _EW

    pltpu.sync_copy(asrc_hbm, asrc_l)
    pltpu.sync_copy(adst_hbm, adst_l)
    pltpu.sync_copy(src_hbm.at[wid], src_l)
    pltpu.sync_copy(dst_hbm.at[wid], dst_l)
    pltpu.sync_copy(ew_hbm.at[pl.ds(base, _EW)], ew_l)
    pltpu.sync_copy(c_hbm, c_l)

    @pl.loop(0, 40)
    def _zero(i):
        zbuf[pl.ds(i * 16, 16)] = jnp.zeros((16,), _f32)

    pltpu.sync_copy(zbuf, denom_sp.at[pl.ds(sid * 624, 640)])
    plsc.subcore_barrier()

    cvec = c_l[...]

    @pl.loop(0, _NQ)
    def _chunk(q):
        @pl.loop(0, _CH // 16)
        def _grp(j):
            p = q * _CH + j * 16
            s16 = src_l[q, pl.ds(j * 16, 16)]
            d16 = dst_l[q, pl.ds(j * 16, 16)]
            al = (plsc.load_gather(asrc_l, [s16])
                  + plsc.load_gather(adst_l, [d16])
                  + ew_l[pl.ds(p, 16)] * cvec)
            al = jnp.maximum(al, 0.2 * al)
            ex_l[pl.ds(p, 16)] = jnp.exp(al)

        pltpu.sync_copy(ex_l.at[pl.ds(q * _CH, _CH)],
                        denom_sp.at[dst_l.at[q]], add=True)

    plsc.subcore_barrier()
    pltpu.sync_copy(ex_l, ex_hbm.at[pl.ds(base, _EW)])
    pltpu.sync_copy(denom_sp.at[pl.ds(sid * 624, 640)], zbuf)
    pltpu.sync_copy(zbuf, dpart_hbm.at[pl.ds(cid * _N + sid * 624, 640)])


def _tc3_body(dp_ref, rd_ref, dinv_ref, sg_ref):
    d = dp_ref[0:1, :] + dp_ref[1:2, :]
    rd = 1.0 / (d + 1e-16)
    rd_ref[...] = rd
    deg = d * rd
    dinv = jnp.where(deg > 0, lax.rsqrt(deg), 0.0)
    dinv_ref[...] = dinv
    sg_ref[...] = dinv * rd


def _tc3(dpart):
    return pl.pallas_call(
        _tc3_body,
        out_shape=[
            jax.ShapeDtypeStruct((1, _N), _f32),
            jax.ShapeDtypeStruct((1, _N), _f32),
            jax.ShapeDtypeStruct((1, _N), _f32),
        ],
    )(dpart)


def _make_rowpass(with_scalars):
    out_type = [jax.ShapeDtypeStruct((_NPAD, _HID), _f32)]
    scratch = [
        pltpu.VMEM((_NQP, _CH), _i32),
        pltpu.VMEM((_NQP, _CH), _i32),
        pltpu.VMEM((1, _CH), _i32),
        pltpu.VMEM((1, _CH), _i32),
        pltpu.VMEM((_CH,), _f32),
        pltpu.VMEM((_CH,), _f32),
        pltpu.VMEM((_NPAD,), _f32),
        pltpu.VMEM((_CH, _HID), _f32),
        pltpu.VMEM((_CH, _HID), _f32),
        pltpu.VMEM((_RC, _HID), _f32),
        pltpu.VMEM_SHARED((_AROWS, _HID), _f32),
        pltpu.SemaphoreType.DMA,
        pltpu.SemaphoreType.DMA,
        pltpu.SemaphoreType.DMA,
        pltpu.SemaphoreType.DMA,
    ]
    if with_scalars:
        out_type += [
            jax.ShapeDtypeStruct((_E,), _f32),
            jax.ShapeDtypeStruct((_E,), _f32),
        ]
        scratch += [
            pltpu.VMEM((_N,), _f32),
            pltpu.VMEM((_CH,), _f32),
            pltpu.VMEM((_CH,), _f32),
        ]

    def body(*refs):
        if with_scalars:
            (src_hbm, dst_hbm, es_hbm, mat_hbm, rscale_hbm, dinv_hbm,
             part_hbm, attn_hbm, w_hbm,
             src_l, dst_l, dstp_a, dstp_b, es_a, es_b, rs_l, rows_a, rows_b,
             obuf, acc_sp, sem0, sem1, ssem0, ssem1,
             dinv_l, attn_s, w_s) = refs
        else:
            (src_hbm, dst_hbm, es_hbm, mat_hbm, rscale_hbm,
             part_hbm,
             src_l, dst_l, dstp_a, dstp_b, es_a, es_b, rs_l, rows_a, rows_b,
             obuf, acc_sp, sem0, sem1, ssem0, ssem1) = refs
        slots = ((rows_a, es_a, dstp_a, sem0, ssem0),
                 (rows_b, es_b, dstp_b, sem1, ssem1))

        def _chunk_copies(q, b):
            rows, es, _, sem, _ = slots[b]
            return (
                pltpu.make_async_copy(mat_hbm.at[src_l.at[q]], rows, sem),
                pltpu.make_async_copy(
                    es_hbm.at[pl.ds(ebase + q * _CH, _CH)], es, sem),
            )

        def _scatter_wait(b):
            rows, _, dstp, _, ssem = slots[b]
            pltpu.make_async_copy(rows, acc_sp.at[dstp.at[0]], ssem).wait()
        cid = lax.axis_index("c")
        sid = lax.axis_index("s")
        ebase = sid * _EP

        pltpu.sync_copy(src_hbm.at[sid], src_l)
        pltpu.sync_copy(dst_hbm.at[sid], dst_l)
        pltpu.sync_copy(rscale_hbm, rs_l)
        if with_scalars:
            pltpu.sync_copy(dinv_hbm, dinv_l)

        if with_scalars:
            @pl.loop(0, _NQP)
            def _esc(q):
                pltpu.sync_copy(es_hbm.at[pl.ds(ebase + q * _CH, _CH)], es_a)

                @pl.loop(0, _CH // 16)
                def _g(j):
                    d16 = dst_l[q, pl.ds(j * 16, 16)]
                    s16 = src_l[q, pl.ds(j * 16, 16)]
                    ex16 = es_a[pl.ds(j * 16, 16)]
                    attn_s[pl.ds(j * 16, 16)] = (
                        ex16 * plsc.load_gather(rs_l, [d16]))
                    w_s[pl.ds(j * 16, 16)] = (
                        ex16 * plsc.load_gather(dinv_l, [s16]))

                @pl.when(cid == 0)
                def _wout():
                    pltpu.sync_copy(
                        attn_s, attn_hbm.at[pl.ds(ebase + q * _CH, _CH)])
                    pltpu.sync_copy(
                        w_s, w_hbm.at[pl.ds(ebase + q * _CH, _CH)])

        rbase0 = sid * _RPT

        for p in range(_NP):
            k = p * _NC + cid
            gbase = k * _NH

            @pl.loop(0, _RC)
            def _zr(i):
                @pl.loop(0, _HID // 16)
                def _zc(j):
                    obuf[i, pl.ds(j * 16, 16)] = jnp.zeros((16,), _f32)

            @pl.loop(0, _NRC)
            def _zero(i):
                pltpu.sync_copy(obuf, acc_sp.at[pl.ds(rbase0 + i * _RC, _RC)])

            @pl.when(sid == 0)
            def _zt():
                pltpu.sync_copy(obuf.at[pl.ds(0, 8)],
                                acc_sp.at[pl.ds(_TRASH, 8)])

            for d in _chunk_copies(0, 0):
                d.start()

            plsc.subcore_barrier()

            @pl.loop(0, _NQP // 2)
            def _g2(g):
                for b in range(2):
                    q = g * 2 + b
                    rows, es, dstp, _, ssem = slots[b]

                    @pl.when(q + 1 < _NQP)
                    def _nxt():
                        @pl.when(q >= 1)
                        def _drain():
                            _scatter_wait(1 - b)

                        for d in _chunk_copies(q + 1, 1 - b):
                            d.start()

                    for d in _chunk_copies(q, b):
                        d.wait()

                    @pl.loop(0, _CH // 16)
                    def _g(j):
                        dd = dst_l[q, pl.ds(j * 16, 16)] - gbase
                        ok = (dd >= 0) & (dd < _NH)
                        dstp[0, pl.ds(j * 16, 16)] = jnp.where(ok, dd, _TRASH)

                    @pl.loop(0, _CH)
                    def _scale(e):
                        sp = plsc.load_gather(es, [jnp.full((16,), e, _i32)])
                        for j in range(_HID // 16):
                            rows[e, pl.ds(j * 16, 16)] = (
                                rows[e, pl.ds(j * 16, 16)] * sp)

                    pltpu.async_copy(rows, acc_sp.at[dstp.at[0]], ssem,
                                     add=True)

            _scatter_wait(0)
            _scatter_wait(1)

            plsc.subcore_barrier()

            @pl.loop(0, _NRC)
            def _out(i):
                rbase = rbase0 + i * _RC
                pltpu.sync_copy(acc_sp.at[pl.ds(rbase, _RC)], obuf)

                @pl.loop(0, _RC)
                def _rs(r):
                    sp = plsc.load_gather(
                        rs_l, [jnp.full((16,), gbase + rbase + r, _i32)])
                    for j in range(_HID // 16):
                        obuf[r, pl.ds(j * 16, 16)] = (
                            obuf[r, pl.ds(j * 16, 16)] * sp)

                pltpu.sync_copy(obuf, part_hbm.at[pl.ds(gbase + rbase, _RC)])

            if p + 1 < _NP:
                plsc.subcore_barrier()

    return pl.kernel(
        body,
        out_type=tuple(out_type) if with_scalars else out_type[0],
        mesh=_sc_mesh,
        compiler_params=_sc_params,
        scratch_types=scratch,
    )


_rowpass_gat = _make_rowpass(True)
_rowpass_gcn = _make_rowpass(False)


def _tcmm_body(p_ref, b_ref, w_ref, bo_ref, o_ref):
    h = jnp.maximum(p_ref[...] + b_ref[...], 0.0)
    o_ref[...] = (jnp.dot(h, w_ref[...], preferred_element_type=_f32)
                  + bo_ref[...])


def _tcmm(p, b, w, bo):
    blk = 2000
    nout = w.shape[1]
    return pl.pallas_call(
        _tcmm_body,
        grid=(_N // blk,),
        in_specs=[
            pl.BlockSpec((blk, _HID), lambda i: (i, 0)),
            pl.BlockSpec((1, _HID), lambda i: (0, 0)),
            pl.BlockSpec((_HID, nout), lambda i: (0, 0)),
            pl.BlockSpec((1, nout), lambda i: (0, 0)),
        ],
        out_specs=pl.BlockSpec((blk, nout), lambda i: (i, 0)),
        out_shape=jax.ShapeDtypeStruct((_N, nout), _f32),
    )(p, b, w, bo)


def kernel(x, edge_index, edge_weight, W_gat, att_src, att_dst, W_edge,
           att_edge, b_gat, W_gcn, b_gcn, W_out, b_out):
    x = x.astype(_f32)
    edge_weight = edge_weight.astype(_f32)

    src32 = edge_index[0].reshape(_NW, _NQ, _CH)
    dst32 = edge_index[1].reshape(_NW, _NQ, _CH)
    src16 = edge_index[0].reshape(_NS, _NQP, _CH)
    dst16 = edge_index[1].reshape(_NS, _NQP, _CH)
    ew = edge_weight.reshape(_E)
    att2 = jnp.concatenate([att_src, att_dst], axis=0).T

    xw, a2, cvec = _tc1(x, W_gat, att2, W_edge, att_edge)
    a_src = a2[:, 0]
    a_dst = a2[:, 1]
    c16 = cvec[0, :16]

    ex, dpart = _sc2(src32, dst32, ew, a_src, a_dst, c16)

    rd2, dinv2, sg2 = _tc3(dpart.reshape(_NC, _N))
    rd = rd2[0]
    dinv = dinv2[0]
    sgcn = sg2[0]

    rd_pad = jnp.pad(rd, (0, _NPAD - _N))
    hpre, attn, w = _rowpass_gat(src16, dst16, ex, xw, rd_pad, dinv)

    hw = _tcmm(hpre[:_N], b_gat.reshape(1, _HID), W_gcn,
               jnp.zeros((1, _HID), _f32))

    sgcn_pad = jnp.pad(sgcn, (0, _NPAD - _N))
    h2pre = _rowpass_gcn(src16, dst16, w, hw, sgcn_pad)

    out = _tcmm(h2pre[:_N], b_gcn.reshape(1, _HID), W_out,
                b_out.reshape(1, -1))

    return (out, attn.reshape(_E, 1))


# speedup vs baseline: 10.3685x; 1.0615x over previous
import dataclasses
import functools

import jax
import jax.numpy as jnp
from jax import lax
from jax.experimental import pallas as pl
from jax.experimental.pallas import tpu as pltpu
from jax.experimental.pallas import tpu_sc as plsc

_N = 10000
_E = 320000
_F = 128
_HID = 128
_HH = _HID // 2
_NC = 2
_NS = 16
_NW = _NC * _NS
_EW = _E // _NW
_CH = 80
_NQ = _EW // _CH
_EP = _E // _NS
_NQP = _EP // _CH
_NP = 2
_NH = 2560
_NPAD = _NP * _NC * _NH
_TRASH = _NH
_AROWS = _NH + 8
_RPT = _NH // _NS
_RC = 16
_NRC = _RPT // _RC

_sc_mesh = plsc.VectorSubcoreMesh(
    core_axis_name="c", subcore_axis_name="s", num_cores=_NC, num_subcores=_NS)

_f32 = jnp.float32
_i32 = jnp.int32

_sc_params = pltpu.CompilerParams()
if "needs_layout_passes" in pltpu.CompilerParams.__dataclass_fields__:
    _sc_params = dataclasses.replace(_sc_params, needs_layout_passes=False)


def _tc1_body(x_ref, wg_ref, att2_ref, we_ref, ae_ref, xw_ref, a2_ref, c_ref):
    xw = jnp.dot(x_ref[...], wg_ref[...], preferred_element_type=_f32)
    xw_ref[...] = xw
    a2_ref[...] = jnp.dot(xw, att2_ref[...], preferred_element_type=_f32)
    c_ref[...] = jnp.full((1, 128), jnp.sum(we_ref[...] * ae_ref[...]), _f32)


def _tc1(x, W_gat, att2, we, ae):
    blk = 2000
    return pl.pallas_call(
        _tc1_body,
        grid=(_N // blk,),
        in_specs=[
            pl.BlockSpec((blk, _F), lambda i: (i, 0)),
            pl.BlockSpec((_F, _HID), lambda i: (0, 0)),
            pl.BlockSpec((_HID, 2), lambda i: (0, 0)),
            pl.BlockSpec((1, _HID), lambda i: (0, 0)),
            pl.BlockSpec((1, _HID), lambda i: (0, 0)),
        ],
        out_specs=[
            pl.BlockSpec((blk, _HID), lambda i: (i, 0)),
            pl.BlockSpec((blk, 2), lambda i: (i, 0)),
            pl.BlockSpec((1, 128), lambda i: (0, 0)),
        ],
        out_shape=[
            jax.ShapeDtypeStruct((_N, _HID), _f32),
            jax.ShapeDtypeStruct((_N, 2), _f32),
            jax.ShapeDtypeStruct((1, 128), _f32),
        ],
    )(x, W_gat, att2, we, ae)


@functools.partial(
    pl.kernel,
    out_type=(
        jax.ShapeDtypeStruct((_E,), _f32),
        jax.ShapeDtypeStruct((_NC * _N,), _f32),
    ),
    mesh=_sc_mesh,
    compiler_params=_sc_params,
    scratch_types=[
        pltpu.VMEM((_N,), _f32),
        pltpu.VMEM((_N,), _f32),
        pltpu.VMEM((_NQ, _CH), _i32),
        pltpu.VMEM((_NQ, _CH), _i32),
        pltpu.VMEM((_EW,), _f32),
        pltpu.VMEM((_EW,), _f32),
        pltpu.VMEM((16,), _f32),
        pltpu.VMEM((640,), _f32),
        pltpu.VMEM_SHARED((_N,), _f32),
    ],
)
def _sc2(src_hbm, dst_hbm, ew_hbm, asrc_hbm, adst_hbm, c_hbm,
         ex_hbm, dpart_hbm,
         asrc_l, adst_l, src_l, dst_l, ew_l, ex_l, c_l, zbuf, denom_sp):
    cid = lax.axis_index("c")
    sid = lax.axis_index("s")
    wid = cid * _NS + sid
    base = wid * _EW

    pltpu.sync_copy(asrc_hbm, asrc_l)
    pltpu.sync_copy(adst_hbm, adst_l)
    pltpu.sync_copy(src_hbm.at[wid], src_l)
    pltpu.sync_copy(dst_hbm.at[wid], dst_l)
    pltpu.sync_copy(ew_hbm.at[pl.ds(base, _EW)], ew_l)
    pltpu.sync_copy(c_hbm, c_l)

    @pl.loop(0, 40)
    def _zero(i):
        zbuf[pl.ds(i * 16, 16)] = jnp.zeros((16,), _f32)

    pltpu.sync_copy(zbuf, denom_sp.at[pl.ds(sid * 624, 640)])
    plsc.subcore_barrier()

    cvec = c_l[...]

    @pl.loop(0, _NQ)
    def _chunk(q):
        @pl.loop(0, _CH // 16)
        def _grp(j):
            p = q * _CH + j * 16
            s16 = src_l[q, pl.ds(j * 16, 16)]
            d16 = dst_l[q, pl.ds(j * 16, 16)]
            al = (plsc.load_gather(asrc_l, [s16])
                  + plsc.load_gather(adst_l, [d16])
                  + ew_l[pl.ds(p, 16)] * cvec)
            al = jnp.maximum(al, 0.2 * al)
            ex_l[pl.ds(p, 16)] = jnp.exp(al)

        pltpu.sync_copy(ex_l.at[pl.ds(q * _CH, _CH)],
                        denom_sp.at[dst_l.at[q]], add=True)

    plsc.subcore_barrier()
    pltpu.sync_copy(ex_l, ex_hbm.at[pl.ds(base, _EW)])
    pltpu.sync_copy(denom_sp.at[pl.ds(sid * 624, 640)], zbuf)
    pltpu.sync_copy(zbuf, dpart_hbm.at[pl.ds(cid * _N + sid * 624, 640)])


def _tc3_body(dp_ref, rd_ref, dinv_ref, sg_ref):
    d = dp_ref[0:1, :] + dp_ref[1:2, :]
    rd = 1.0 / (d + 1e-16)
    rd_ref[...] = rd
    deg = d * rd
    dinv = jnp.where(deg > 0, lax.rsqrt(deg), 0.0)
    dinv_ref[...] = dinv
    sg_ref[...] = dinv * rd


def _tc3(dpart):
    return pl.pallas_call(
        _tc3_body,
        out_shape=[
            jax.ShapeDtypeStruct((1, _N), _f32),
            jax.ShapeDtypeStruct((1, _N), _f32),
            jax.ShapeDtypeStruct((1, _N), _f32),
        ],
    )(dpart)


def _make_rowpass(with_scalars):
    out_type = [jax.ShapeDtypeStruct((_NPAD, _HID), _f32)]
    scratch = [
        pltpu.VMEM((_NQP, _CH), _i32),
        pltpu.VMEM((_NQP, _CH), _i32),
        pltpu.VMEM((1, _CH), _i32),
        pltpu.VMEM((1, _CH), _i32),
        pltpu.VMEM((_CH,), _f32),
        pltpu.VMEM((_CH,), _f32),
        pltpu.VMEM((_NPAD,), _f32),
        pltpu.VMEM((_CH, _HID), _f32),
        pltpu.VMEM((_CH, _HID), _f32),
        pltpu.VMEM((_RC, _HID), _f32),
        pltpu.VMEM_SHARED((_AROWS, _HID), _f32),
        pltpu.SemaphoreType.DMA,
        pltpu.SemaphoreType.DMA,
        pltpu.SemaphoreType.DMA,
        pltpu.SemaphoreType.DMA,
    ]
    if with_scalars:
        out_type += [
            jax.ShapeDtypeStruct((_E,), _f32),
            jax.ShapeDtypeStruct((_E,), _f32),
        ]
        scratch += [
            pltpu.VMEM((_N,), _f32),
            pltpu.VMEM((400,), _f32),
            pltpu.VMEM((400,), _f32),
            pltpu.VMEM((400,), _f32),
        ]

    def body(*refs):
        if with_scalars:
            (src_hbm, dst_hbm, es_hbm, mat_hbm, rscale_hbm, dinv_hbm,
             part_hbm, attn_hbm, w_hbm,
             src_l, dst_l, dstp_a, dstp_b, es_a, es_b, rs_l, rows_a, rows_b,
             obuf, acc_sp, sem0, sem1, ssem0, ssem1,
             dinv_l, exb_s, attn_s, w_s) = refs
        else:
            (src_hbm, dst_hbm, es_hbm, mat_hbm, rscale_hbm,
             part_hbm,
             src_l, dst_l, dstp_a, dstp_b, es_a, es_b, rs_l, rows_a, rows_b,
             obuf, acc_sp, sem0, sem1, ssem0, ssem1) = refs
        slots = ((rows_a, es_a, dstp_a, sem0, ssem0),
                 (rows_b, es_b, dstp_b, sem1, ssem1))

        def _chunk_copies(q, b):
            rows, es, _, sem, _ = slots[b]
            return (
                pltpu.make_async_copy(mat_hbm.at[src_l.at[q]], rows, sem),
                pltpu.make_async_copy(
                    es_hbm.at[pl.ds(ebase + q * _CH, _CH)], es, sem),
            )

        def _scatter_wait(b):
            rows, _, dstp, _, ssem = slots[b]
            pltpu.make_async_copy(rows, acc_sp.at[dstp.at[0]], ssem).wait()
        cid = lax.axis_index("c")
        sid = lax.axis_index("s")
        ebase = sid * _EP

        pltpu.sync_copy(src_hbm.at[sid], src_l)
        pltpu.sync_copy(dst_hbm.at[sid], dst_l)
        pltpu.sync_copy(rscale_hbm, rs_l)
        if with_scalars:
            pltpu.sync_copy(dinv_hbm, dinv_l)

        if with_scalars:
            @pl.loop(0, _NQP // 5)
            def _esc(t):
                pltpu.sync_copy(es_hbm.at[pl.ds(ebase + t * 400, 400)], exb_s)

                @pl.loop(0, 5)
                def _c(c):
                    q = t * 5 + c

                    @pl.loop(0, _CH // 16)
                    def _g(j):
                        d16 = dst_l[q, pl.ds(j * 16, 16)]
                        s16 = src_l[q, pl.ds(j * 16, 16)]
                        o = c * _CH + j * 16
                        ex16 = exb_s[pl.ds(o, 16)]
                        attn_s[pl.ds(o, 16)] = (
                            ex16 * plsc.load_gather(rs_l, [d16]))
                        w_s[pl.ds(o, 16)] = (
                            ex16 * plsc.load_gather(dinv_l, [s16]))

                @pl.when(cid == 0)
                def _wout():
                    pltpu.sync_copy(
                        attn_s, attn_hbm.at[pl.ds(ebase + t * 400, 400)])
                    pltpu.sync_copy(
                        w_s, w_hbm.at[pl.ds(ebase + t * 400, 400)])

        rbase0 = sid * _RPT

        for p in range(_NP):
            k = p * _NC + cid
            gbase = k * _NH

            @pl.loop(0, _RC)
            def _zr(i):
                @pl.loop(0, _HID // 16)
                def _zc(j):
                    obuf[i, pl.ds(j * 16, 16)] = jnp.zeros((16,), _f32)

            @pl.loop(0, _NRC)
            def _zero(i):
                pltpu.sync_copy(obuf, acc_sp.at[pl.ds(rbase0 + i * _RC, _RC)])

            @pl.when(sid == 0)
            def _zt():
                pltpu.sync_copy(obuf.at[pl.ds(0, 8)],
                                acc_sp.at[pl.ds(_TRASH, 8)])

            for d in _chunk_copies(0, 0):
                d.start()

            plsc.subcore_barrier()

            @pl.loop(0, _NQP // 2)
            def _g2(g):
                for b in range(2):
                    q = g * 2 + b
                    rows, es, dstp, _, ssem = slots[b]

                    @pl.when(q + 1 < _NQP)
                    def _nxt():
                        @pl.when(q >= 1)
                        def _drain():
                            _scatter_wait(1 - b)

                        for d in _chunk_copies(q + 1, 1 - b):
                            d.start()

                    for d in _chunk_copies(q, b):
                        d.wait()

                    @pl.loop(0, _CH // 16)
                    def _g(j):
                        dd = dst_l[q, pl.ds(j * 16, 16)] - gbase
                        ok = (dd >= 0) & (dd < _NH)
                        dstp[0, pl.ds(j * 16, 16)] = jnp.where(ok, dd, _TRASH)

                    @pl.loop(0, _CH)
                    def _scale(e):
                        sp = plsc.load_gather(es, [jnp.full((16,), e, _i32)])
                        for j in range(_HID // 16):
                            rows[e, pl.ds(j * 16, 16)] = (
                                rows[e, pl.ds(j * 16, 16)] * sp)

                    pltpu.async_copy(rows, acc_sp.at[dstp.at[0]], ssem,
                                     add=True)

            _scatter_wait(0)
            _scatter_wait(1)

            plsc.subcore_barrier()

            @pl.loop(0, _NRC)
            def _out(i):
                rbase = rbase0 + i * _RC
                pltpu.sync_copy(acc_sp.at[pl.ds(rbase, _RC)], obuf)

                @pl.loop(0, _RC)
                def _rs(r):
                    sp = plsc.load_gather(
                        rs_l, [jnp.full((16,), gbase + rbase + r, _i32)])
                    for j in range(_HID // 16):
                        obuf[r, pl.ds(j * 16, 16)] = (
                            obuf[r, pl.ds(j * 16, 16)] * sp)

                pltpu.sync_copy(obuf, part_hbm.at[pl.ds(gbase + rbase, _RC)])

            if p + 1 < _NP:
                plsc.subcore_barrier()

    return pl.kernel(
        body,
        out_type=tuple(out_type) if with_scalars else out_type[0],
        mesh=_sc_mesh,
        compiler_params=_sc_params,
        scratch_types=scratch,
    )


_rowpass_gat = _make_rowpass(True)
_rowpass_gcn = _make_rowpass(False)


def _tcmm_body(p_ref, b_ref, w_ref, bo_ref, o_ref):
    h = jnp.maximum(p_ref[...] + b_ref[...], 0.0)
    o_ref[...] = (jnp.dot(h, w_ref[...], preferred_element_type=_f32)
                  + bo_ref[...])


def _tcmm(p, b, w, bo):
    blk = 2000
    nout = w.shape[1]
    return pl.pallas_call(
        _tcmm_body,
        grid=(_N // blk,),
        in_specs=[
            pl.BlockSpec((blk, _HID), lambda i: (i, 0)),
            pl.BlockSpec((1, _HID), lambda i: (0, 0)),
            pl.BlockSpec((_HID, nout), lambda i: (0, 0)),
            pl.BlockSpec((1, nout), lambda i: (0, 0)),
        ],
        out_specs=pl.BlockSpec((blk, nout), lambda i: (i, 0)),
        out_shape=jax.ShapeDtypeStruct((_N, nout), _f32),
    )(p, b, w, bo)


def kernel(x, edge_index, edge_weight, W_gat, att_src, att_dst, W_edge,
           att_edge, b_gat, W_gcn, b_gcn, W_out, b_out):
    x = x.astype(_f32)
    edge_weight = edge_weight.astype(_f32)

    src32 = edge_index[0].reshape(_NW, _NQ, _CH)
    dst32 = edge_index[1].reshape(_NW, _NQ, _CH)
    src16 = edge_index[0].reshape(_NS, _NQP, _CH)
    dst16 = edge_index[1].reshape(_NS, _NQP, _CH)
    ew = edge_weight.reshape(_E)
    att2 = jnp.concatenate([att_src, att_dst], axis=0).T

    xw, a2, cvec = _tc1(x, W_gat, att2, W_edge, att_edge)
    a_src = a2[:, 0]
    a_dst = a2[:, 1]
    c16 = cvec[0, :16]

    ex, dpart = _sc2(src32, dst32, ew, a_src, a_dst, c16)

    rd2, dinv2, sg2 = _tc3(dpart.reshape(_NC, _N))
    rd = rd2[0]
    dinv = dinv2[0]
    sgcn = sg2[0]

    rd_pad = jnp.pad(rd, (0, _NPAD - _N))
    hpre, attn, w = _rowpass_gat(src16, dst16, ex, xw, rd_pad, dinv)

    hw = _tcmm(hpre[:_N], b_gat.reshape(1, _HID), W_gcn,
               jnp.zeros((1, _HID), _f32))

    sgcn_pad = jnp.pad(sgcn, (0, _NPAD - _N))
    h2pre = _rowpass_gcn(src16, dst16, w, hw, sgcn_pad)

    out = _tcmm(h2pre[:_N], b_gcn.reshape(1, _HID), W_out,
                b_out.reshape(1, -1))

    return (out, attn.reshape(_E, 1))
